# baseline (speedup 1.0000x reference)
import functools
import math

import jax
import jax.numpy as jnp
from jax import lax
from jax.experimental import pallas as pl
from jax.experimental.pallas import tpu as pltpu
from jax.experimental.pallas import tpu_sc as plsc

VOCAB = 1000000
D = 64
BATCH = 4096
HIST = 200
SCALE = math.sqrt(D)
NBUF = 2
VCHUNK = 32768


def _relayout_body(tt_ref, t1_ref):
    t1_ref[:, 0:D] = tt_ref[...].T


def _relayout(tt):
    return pl.pallas_call(
        _relayout_body,
        grid=((VOCAB + VCHUNK - 1) // VCHUNK,),
        in_specs=[pl.BlockSpec((D, VCHUNK), lambda c: (0, c))],
        out_specs=pl.BlockSpec((VCHUNK, 128), lambda c: (c, 0)),
        out_shape=jax.ShapeDtypeStruct((VOCAB, 128), jnp.float32),
    )(tt)


@jax.jit
def _embed(xt, t1):
    info = plsc.get_sparse_core_info()
    nw = info.num_cores * info.num_subcores
    hb = BATCH // nw

    mesh = plsc.VectorSubcoreMesh(core_axis_name="c", subcore_axis_name="s")

    @functools.partial(
        pl.kernel,
        mesh=mesh,
        out_type=jax.ShapeDtypeStruct((HIST, D, BATCH), jnp.float32),
        compiler_params=pltpu.CompilerParams(needs_layout_passes=False),
        scratch_types=[
            pltpu.VMEM((HIST, hb), jnp.int32),
            pltpu.VMEM((hb,), jnp.int32),
            pltpu.VMEM((hb,), jnp.int32),
            pltpu.VMEM((hb, 128), jnp.float32),
            pltpu.VMEM((hb, 128), jnp.float32),
            pltpu.VMEM((D, hb), jnp.float32),
            pltpu.VMEM((D, hb), jnp.float32),
            pltpu.SemaphoreType.DMA,
            pltpu.SemaphoreType.DMA,
            pltpu.SemaphoreType.DMA,
            pltpu.SemaphoreType.DMA,
        ],
    )
    def k(xt_hbm, t1_hbm, out_hbm, idx_all, vr0, vr1,
          g0, g1, s0, s1,
          gsem0, gsem1, osem0, osem1):
        vr = (vr0, vr1)
        gbuf = (g0, g1)
        sbuf = (s0, s1)
        gsem = (gsem0, gsem1)
        osem = (osem0, osem1)

        wid = lax.axis_index("s") * info.num_cores + lax.axis_index("c")
        bbase = wid * hb

        pltpu.sync_copy(xt_hbm.at[:, pl.ds(bbase, hb)], idx_all)

        def start_gather(h, s):
            for q in range(hb // 16):
                vr[s][pl.ds(16 * q, 16)] = idx_all[h, pl.ds(16 * q, 16)]
            pltpu.async_copy(t1_hbm.at[vr[s]], gbuf[s], gsem[s])

        for s in range(NBUF):
            start_gather(s, s)

        lanes = lax.iota(jnp.int32, 16)
        rot = [(lanes + k) & 15 for k in range(16)]

        @pl.loop(0, HIST, step=NBUF)
        def outer(grp):
            for s in range(NBUF):
                cur = grp + s
                pltpu.make_async_copy(
                    t1_hbm.at[vr[s]], gbuf[s], gsem[s]).wait()

                @pl.when(cur >= NBUF)
                def _():
                    pltpu.make_async_copy(
                        sbuf[s], out_hbm.at[0, :, pl.ds(bbase, hb)],
                        osem[s]).wait()

                @plsc.parallel_loop(0, hb // 16, unroll=2)
                def tr(jb):
                    rowg = jb * 16 + lanes
                    for db in range(D // 16):
                        for kk in range(16):
                            rs = rot[kk] + db * 16
                            v = plsc.load_gather(gbuf[s], [rowg, rs])
                            plsc.store_scatter(
                                sbuf[s], [rs, rowg], v * SCALE)

                pltpu.async_copy(
                    sbuf[s], out_hbm.at[cur, :, pl.ds(bbase, hb)], osem[s])

                @pl.when(cur + NBUF < HIST)
                def _():
                    start_gather(cur + NBUF, s)

        for s in range(NBUF):
            pltpu.make_async_copy(
                sbuf[s], out_hbm.at[0, :, pl.ds(bbase, hb)], osem[s]).wait()

    return k(xt, t1)


def kernel(x, table):
    xt = x.T.astype(jnp.int32)
    tt = table.T
    t1 = _relayout(tt)
    out3 = _embed(xt, t1)
    return out3.transpose(2, 0, 1)


# speedup vs baseline: 1.4508x; 1.4508x over previous
import functools
import math

import jax
import jax.numpy as jnp
from jax import lax
from jax.experimental import pallas as pl
from jax.experimental.pallas import tpu as pltpu
from jax.experimental.pallas import tpu_sc as plsc

VOCAB = 1000000
D = 64
BATCH = 4096
HIST = 200
SCALE = math.sqrt(D)
NBUF = 2
VCHUNK = 32768


def _relayout_body(tt_ref, t1_ref):
    t1_ref[:, 0:D] = tt_ref[...].T


def _relayout(tt):
    return pl.pallas_call(
        _relayout_body,
        grid=((VOCAB + VCHUNK - 1) // VCHUNK,),
        in_specs=[pl.BlockSpec((D, VCHUNK), lambda c: (0, c))],
        out_specs=pl.BlockSpec((VCHUNK, 128), lambda c: (c, 0)),
        out_shape=jax.ShapeDtypeStruct((VOCAB, 128), jnp.float32),
    )(tt)


@jax.jit
def _embed(xt, t1):
    info = plsc.get_sparse_core_info()
    nw = info.num_cores * info.num_subcores
    hb = BATCH // nw

    mesh = plsc.VectorSubcoreMesh(core_axis_name="c", subcore_axis_name="s")

    @functools.partial(
        pl.kernel,
        mesh=mesh,
        out_type=jax.ShapeDtypeStruct((HIST, D, BATCH), jnp.float32),
        compiler_params=pltpu.CompilerParams(needs_layout_passes=False),
        scratch_types=[
            pltpu.VMEM((HIST, hb), jnp.int32),
            pltpu.VMEM((hb,), jnp.int32),
            pltpu.VMEM((hb,), jnp.int32),
            pltpu.VMEM((hb, 128), jnp.float32),
            pltpu.VMEM((hb, 128), jnp.float32),
            pltpu.VMEM((D, hb), jnp.float32),
            pltpu.VMEM((D, hb), jnp.float32),
            pltpu.SemaphoreType.DMA,
            pltpu.SemaphoreType.DMA,
            pltpu.SemaphoreType.DMA,
            pltpu.SemaphoreType.DMA,
        ],
    )
    def k(xt_hbm, t1_hbm, out_hbm, idx_all, vr0, vr1,
          g0, g1, s0, s1,
          gsem0, gsem1, osem0, osem1):
        vr = (vr0, vr1)
        gbuf = (g0, g1)
        sbuf = (s0, s1)
        gsem = (gsem0, gsem1)
        osem = (osem0, osem1)

        wid = lax.axis_index("s") * info.num_cores + lax.axis_index("c")
        bbase = wid * hb

        pltpu.sync_copy(xt_hbm.at[:, pl.ds(bbase, hb)], idx_all)

        def start_gather(h, s):
            for q in range(hb // 16):
                vr[s][pl.ds(16 * q, 16)] = idx_all[h, pl.ds(16 * q, 16)]
            pltpu.async_copy(t1_hbm.at[vr[s]], gbuf[s], gsem[s])

        for s in range(NBUF):
            start_gather(s, s)

        lanes = lax.iota(jnp.int32, 16)
        rot = [(lanes + k) & 15 for k in range(16)]

        @pl.loop(0, HIST, step=NBUF)
        def outer(grp):
            for s in range(NBUF):
                cur = grp + s
                pltpu.make_async_copy(
                    t1_hbm.at[vr[s]], gbuf[s], gsem[s]).wait()

                @pl.when(cur >= NBUF)
                def _():
                    pltpu.make_async_copy(
                        sbuf[s], out_hbm.at[0, :, pl.ds(bbase, hb)],
                        osem[s]).wait()

                @plsc.parallel_loop(0, hb // 16)
                def tr(jb):
                    rowg = jb * 16 + lanes
                    for db in range(D // 16):
                        for kk in range(16):
                            rs = rot[kk] + db * 16
                            v = plsc.load_gather(gbuf[s], [rowg, rs])
                            plsc.store_scatter(
                                sbuf[s], [rs, rowg], v * SCALE)

                pltpu.async_copy(
                    sbuf[s], out_hbm.at[cur, :, pl.ds(bbase, hb)], osem[s])

                @pl.when(cur + NBUF < HIST)
                def _():
                    start_gather(cur + NBUF, s)

        for s in range(NBUF):
            pltpu.make_async_copy(
                sbuf[s], out_hbm.at[0, :, pl.ds(bbase, hb)], osem[s]).wait()

    return k(xt, t1)


def kernel(x, table):
    xt = x.T.astype(jnp.int32)
    tt = table.T
    t1 = _relayout(tt)
    out3 = _embed(xt, t1)
    return out3.transpose(2, 0, 1)
